# baseline (speedup 1.0000x reference)
import functools

import jax
import jax.numpy as jnp
from jax import lax
from jax.experimental import pallas as pl
from jax.experimental.pallas import tpu as pltpu
from jax.experimental.pallas import tpu_sc as plsc

N = 10000
E = 320000
DIN = 128
DH = 64
DHH = DH // 2
DOUT = 128
G = 64

NC = 2
NS = 16
K = 128
CHE = 160
E_PAD = NS * CHE * K
NBUF = 10
NH = 2
NGRP = CHE // NBUF
CHD = CHE // NC
N_PAD = 10112
PAD_ROW = N
RPT = N_PAD // NS
BLK = 2528

_mesh = plsc.VectorSubcoreMesh(
    core_axis_name="c", subcore_axis_name="s", num_cores=NC, num_subcores=NS
)


@functools.partial(
    pl.kernel,
    out_type=jax.ShapeDtypeStruct((NC, N_PAD, DHH), jnp.float32),
    mesh=_mesh,
    scratch_types=[
        pltpu.VMEM((CHE, K), jnp.int32),
        pltpu.VMEM((CHE, K), jnp.int32),
        pltpu.VMEM((NBUF, K, DHH), jnp.float32),
        pltpu.VMEM_SHARED((N_PAD, DHH), jnp.float32),
        pltpu.VMEM_SHARED((N_PAD, DHH), jnp.float32),
    ]
    + [pltpu.SemaphoreType.DMA] * (2 * NBUF),
    compiler_params=pltpu.CompilerParams(use_tc_tiling_on_sc=False),
)
def _sc_edge(u_hbm, srcs_hbm, dsts_hbm, out_hbm, sidx, didx, buf, u_s, acc, *sems):
    gsems = sems[:NBUF]
    ssems = sems[NBUF:]
    cid = lax.axis_index("c")
    sid = lax.axis_index("s")
    pltpu.sync_copy(srcs_hbm.at[sid], sidx)
    pltpu.sync_copy(dsts_hbm.at[sid], didx)
    r0 = sid * RPT

    pltpu.sync_copy(u_hbm.at[cid, pl.ds(r0, RPT)], u_s.at[pl.ds(r0, RPT)])
    pltpu.sync_copy(u_hbm.at[cid, pl.ds(r0, RPT)], acc.at[pl.ds(r0, RPT)])
    plsc.subcore_barrier()

    def gsrc(b):
        return u_hbm.at[cid] if b < NH else u_s

    for b in range(NBUF):
        pltpu.async_copy(gsrc(b).at[sidx.at[b]], buf.at[b], gsems[b])

    order = list(range(NH, NBUF)) + list(range(NH))

    def group(g, carry):
        for b in order:
            c = g * NBUF + b
            pltpu.make_async_copy(gsrc(b).at[sidx.at[c]], buf.at[b], gsems[b]).wait()
            pltpu.async_copy(buf.at[b], acc.at[didx.at[c]], ssems[b], add=True)
        for b in order:
            c = g * NBUF + b
            pltpu.make_async_copy(buf.at[b], acc.at[didx.at[c]], ssems[b]).wait()

            @pl.when(g < NGRP - 1)
            def _():
                pltpu.async_copy(gsrc(b).at[sidx.at[c + NBUF]], buf.at[b], gsems[b])

        return carry

    lax.fori_loop(0, NGRP, group, 0)
    plsc.subcore_barrier()
    pltpu.sync_copy(acc.at[pl.ds(r0, RPT)], out_hbm.at[cid, pl.ds(r0, RPT)])


@functools.partial(
    pl.kernel,
    out_type=jax.ShapeDtypeStruct((NC, N_PAD, 16), jnp.float32),
    mesh=_mesh,
    scratch_types=[
        pltpu.VMEM((CHD, K), jnp.int32),
        pltpu.VMEM((K, 16), jnp.float32),
        pltpu.VMEM_SHARED((N_PAD, 16), jnp.float32),
    ]
    + [pltpu.SemaphoreType.DMA] * NBUF,
    compiler_params=pltpu.CompilerParams(use_tc_tiling_on_sc=False),
)
def _sc_deg(dsts_hbm, ones_hbm, zeros_hbm, out_hbm, didx, ones_v, acc, *ssems):
    cid = lax.axis_index("c")
    sid = lax.axis_index("s")
    pltpu.sync_copy(dsts_hbm.at[sid, pl.ds(cid * CHD, CHD)], didx)
    pltpu.sync_copy(ones_hbm, ones_v)
    r0 = sid * RPT
    pltpu.sync_copy(zeros_hbm.at[pl.ds(r0, RPT)], acc.at[pl.ds(r0, RPT)])
    plsc.subcore_barrier()

    for b in range(NBUF):
        pltpu.async_copy(ones_v, acc.at[didx.at[b]], ssems[b], add=True)

    def group(g, carry):
        for b in range(NBUF):
            c = g * NBUF + b
            pltpu.make_async_copy(ones_v, acc.at[didx.at[c]], ssems[b]).wait()

            @pl.when(g < CHD // NBUF - 1)
            def _():
                pltpu.async_copy(ones_v, acc.at[didx.at[c + NBUF]], ssems[b], add=True)

        return carry

    lax.fori_loop(0, CHD // NBUF, group, 0)
    plsc.subcore_barrier()
    pltpu.sync_copy(acc.at[pl.ds(r0, RPT)], out_hbm.at[cid, pl.ds(r0, RPT)])


def _tc_a_body(x_ref, dega_ref, w_ref, u_ref, dinv_ref):
    deg = dega_ref[0] + dega_ref[1] + 1.0
    dv = lax.rsqrt(deg)
    dinv_ref[...] = dv
    u = dv[:, 0:1] * jnp.dot(x_ref[...], w_ref[...], preferred_element_type=jnp.float32)
    u_ref[0] = u[:, :DHH]
    u_ref[1] = u[:, DHH:]


_tc_a = pl.pallas_call(
    _tc_a_body,
    grid=(N_PAD // BLK,),
    in_specs=[
        pl.BlockSpec((BLK, DIN), lambda i: (i, 0)),
        pl.BlockSpec((NC, BLK, 16), lambda i: (0, i, 0)),
        pl.BlockSpec((DIN, DH), lambda i: (0, 0)),
    ],
    out_specs=[
        pl.BlockSpec((NC, BLK, DHH), lambda i: (0, i, 0)),
        pl.BlockSpec((BLK, 16), lambda i: (i, 0)),
    ],
    out_shape=[
        jax.ShapeDtypeStruct((NC, N_PAD, DHH), jnp.float32),
        jax.ShapeDtypeStruct((N_PAD, 16), jnp.float32),
    ],
)


def _tc_b_body(acc_ref, dinv_ref, w_ref, b_ref, un_ref):
    dv = dinv_ref[...][:, 0:1]
    s = jnp.concatenate([acc_ref[0], acc_ref[1]], axis=1)
    t = jnp.maximum(dv * s + b_ref[...], 0.0)
    u = dv * jnp.dot(t, w_ref[...], preferred_element_type=jnp.float32)
    un_ref[0] = u[:, :DHH]
    un_ref[1] = u[:, DHH:]


_tc_b = pl.pallas_call(
    _tc_b_body,
    grid=(N_PAD // BLK,),
    in_specs=[
        pl.BlockSpec((NC, BLK, DHH), lambda i: (0, i, 0)),
        pl.BlockSpec((BLK, 16), lambda i: (i, 0)),
        pl.BlockSpec((DH, DH), lambda i: (0, 0)),
        pl.BlockSpec((1, DH), lambda i: (0, 0)),
    ],
    out_specs=pl.BlockSpec((NC, BLK, DHH), lambda i: (0, i, 0)),
    out_shape=jax.ShapeDtypeStruct((NC, N_PAD, DHH), jnp.float32),
)


def _tc_f_body(acc_ref, dinv_ref, b_ref, batch_ref, wlin_ref, blin_ref, out_ref, sums_ref, cnt_ref):
    i = pl.program_id(0)

    @pl.when(i == 0)
    def _():
        sums_ref[...] = jnp.zeros_like(sums_ref)
        cnt_ref[...] = jnp.zeros_like(cnt_ref)

    dv = dinv_ref[...][:, 0:1]
    s = jnp.concatenate([acc_ref[0], acc_ref[1]], axis=1)
    t = jnp.maximum(dv * s + b_ref[...], 0.0)
    p = (batch_ref[...] == lax.broadcasted_iota(jnp.int32, (BLK, G), 1)).astype(
        jnp.float32
    )
    sums_ref[...] += lax.dot_general(
        p, t, (((0,), (0,)), ((), ())), preferred_element_type=jnp.float32
    )
    cnt_ref[...] += lax.dot_general(
        p,
        jnp.ones((BLK, DH), jnp.float32),
        (((0,), (0,)), ((), ())),
        preferred_element_type=jnp.float32,
    )

    @pl.when(i == pl.num_programs(0) - 1)
    def _():
        pooled = sums_ref[...] / jnp.maximum(cnt_ref[...], 1.0)
        out_ref[...] = (
            jnp.dot(pooled, wlin_ref[...], preferred_element_type=jnp.float32)
            + blin_ref[...]
        )


_tc_f = pl.pallas_call(
    _tc_f_body,
    grid=(N_PAD // BLK,),
    in_specs=[
        pl.BlockSpec((NC, BLK, DHH), lambda i: (0, i, 0)),
        pl.BlockSpec((BLK, 16), lambda i: (i, 0)),
        pl.BlockSpec((1, DH), lambda i: (0, 0)),
        pl.BlockSpec((BLK, 1), lambda i: (i, 0)),
        pl.BlockSpec((DH, DOUT), lambda i: (0, 0)),
        pl.BlockSpec((1, DOUT), lambda i: (0, 0)),
    ],
    out_specs=pl.BlockSpec((G, DOUT), lambda i: (0, 0)),
    out_shape=jax.ShapeDtypeStruct((G, DOUT), jnp.float32),
    scratch_shapes=[
        pltpu.VMEM((G, DH), jnp.float32),
        pltpu.VMEM((G, DH), jnp.float32),
    ],
)


def kernel(x, edge_index, batch, W1, b1, Wh, bh, Wlin, blin):
    src = edge_index[0]
    dst = edge_index[1]
    srcs = jnp.full((E_PAD,), PAD_ROW, jnp.int32).at[:E].set(src).reshape(NS, CHE, K)
    dsts = jnp.full((E_PAD,), PAD_ROW, jnp.int32).at[:E].set(dst).reshape(NS, CHE, K)
    zeros16 = jnp.zeros((N_PAD, 16), jnp.float32)
    ones16 = jnp.ones((K, 16), jnp.float32)
    x_p = jnp.zeros((N_PAD, DIN), jnp.float32).at[:N].set(x)
    batch_p = jnp.full((N_PAD, 1), G, jnp.int32).at[:N, 0].set(batch)

    dega = _sc_deg(dsts, ones16, zeros16)
    u, dinv = _tc_a(x_p, dega, W1)
    biases = [b1] + [bh[i] for i in range(8)]
    for i in range(9):
        acc = _sc_edge(u, srcs, dsts)
        u = _tc_b(acc, dinv, Wh[i], biases[i].reshape(1, DH))
    acc = _sc_edge(u, srcs, dsts)
    return _tc_f(
        acc, dinv, bh[8].reshape(1, DH), batch_p, Wlin, blin.reshape(1, DOUT)
    )


# speedup vs baseline: 1.0946x; 1.0946x over previous
import functools

import jax
import jax.numpy as jnp
from jax import lax
from jax.experimental import pallas as pl
from jax.experimental.pallas import tpu as pltpu
from jax.experimental.pallas import tpu_sc as plsc

N = 10000
E = 320000
DIN = 128
DH = 64
DHH = DH // 2
DOUT = 128
G = 64

NC = 2
NS = 16
K = 128
CHE = 160
E_PAD = NS * CHE * K
NBUF = 10
NH = 2
NGRP = CHE // NBUF
CHD = CHE // NC
N_PAD = 10112
PAD_ROW = N
RPT = N_PAD // NS
BLK = 2528

_mesh = plsc.VectorSubcoreMesh(
    core_axis_name="c", subcore_axis_name="s", num_cores=NC, num_subcores=NS
)


@functools.partial(
    pl.kernel,
    out_type=jax.ShapeDtypeStruct((NC, N_PAD, DHH), jnp.float32),
    mesh=_mesh,
    scratch_types=[
        pltpu.VMEM((CHE, K), jnp.int32),
        pltpu.VMEM((CHE, K), jnp.int32),
        pltpu.VMEM((NBUF, K, DHH), jnp.float32),
        pltpu.VMEM_SHARED((N_PAD, DHH), jnp.float32),
        pltpu.VMEM_SHARED((N_PAD, DHH), jnp.float32),
    ]
    + [pltpu.SemaphoreType.DMA] * (2 * NBUF),
    compiler_params=pltpu.CompilerParams(use_tc_tiling_on_sc=False),
)
def _sc_edge(u_hbm, srcs_hbm, dsts_hbm, out_hbm, sidx, didx, buf, u_s, acc, *sems):
    gsems = sems[:NBUF]
    ssems = sems[NBUF:]
    cid = lax.axis_index("c")
    sid = lax.axis_index("s")
    pltpu.sync_copy(srcs_hbm.at[sid], sidx)
    pltpu.sync_copy(dsts_hbm.at[sid], didx)
    r0 = sid * RPT

    pltpu.sync_copy(u_hbm.at[cid, pl.ds(r0, RPT)], u_s.at[pl.ds(r0, RPT)])
    pltpu.sync_copy(u_hbm.at[cid, pl.ds(r0, RPT)], acc.at[pl.ds(r0, RPT)])
    plsc.subcore_barrier()

    def gsrc(b):
        return u_hbm.at[cid] if b < NH else u_s

    for b in range(NBUF):
        pltpu.async_copy(gsrc(b).at[sidx.at[b]], buf.at[b], gsems[b])

    def group(g, carry):
        for b in range(NBUF):
            c = g * NBUF + b
            pltpu.make_async_copy(gsrc(b).at[sidx.at[c]], buf.at[b], gsems[b]).wait()
            pltpu.async_copy(buf.at[b], acc.at[didx.at[c]], ssems[b], add=True)
        for b in range(NBUF):
            c = g * NBUF + b
            pltpu.make_async_copy(buf.at[b], acc.at[didx.at[c]], ssems[b]).wait()

            @pl.when(g < NGRP - 1)
            def _():
                pltpu.async_copy(gsrc(b).at[sidx.at[c + NBUF]], buf.at[b], gsems[b])

        return carry

    lax.fori_loop(0, NGRP, group, 0)
    plsc.subcore_barrier()
    pltpu.sync_copy(acc.at[pl.ds(r0, RPT)], out_hbm.at[cid, pl.ds(r0, RPT)])


@functools.partial(
    pl.kernel,
    out_type=jax.ShapeDtypeStruct((NC, N_PAD, 16), jnp.float32),
    mesh=_mesh,
    scratch_types=[
        pltpu.VMEM((CHD, K), jnp.int32),
        pltpu.VMEM((K, 16), jnp.float32),
        pltpu.VMEM_SHARED((N_PAD, 16), jnp.float32),
    ]
    + [pltpu.SemaphoreType.DMA] * NBUF,
    compiler_params=pltpu.CompilerParams(use_tc_tiling_on_sc=False),
)
def _sc_deg(dsts_hbm, ones_hbm, zeros_hbm, out_hbm, didx, ones_v, acc, *ssems):
    cid = lax.axis_index("c")
    sid = lax.axis_index("s")
    pltpu.sync_copy(dsts_hbm.at[sid, pl.ds(cid * CHD, CHD)], didx)
    pltpu.sync_copy(ones_hbm, ones_v)
    r0 = sid * RPT
    pltpu.sync_copy(zeros_hbm.at[pl.ds(r0, RPT)], acc.at[pl.ds(r0, RPT)])
    plsc.subcore_barrier()

    for b in range(NBUF):
        pltpu.async_copy(ones_v, acc.at[didx.at[b]], ssems[b], add=True)

    def group(g, carry):
        for b in range(NBUF):
            c = g * NBUF + b
            pltpu.make_async_copy(ones_v, acc.at[didx.at[c]], ssems[b]).wait()

            @pl.when(g < CHD // NBUF - 1)
            def _():
                pltpu.async_copy(ones_v, acc.at[didx.at[c + NBUF]], ssems[b], add=True)

        return carry

    lax.fori_loop(0, CHD // NBUF, group, 0)
    plsc.subcore_barrier()
    pltpu.sync_copy(acc.at[pl.ds(r0, RPT)], out_hbm.at[cid, pl.ds(r0, RPT)])


def _tc_a_body(x_ref, dega_ref, w_ref, u_ref, dinv_ref):
    deg = dega_ref[0] + dega_ref[1] + 1.0
    dv = lax.rsqrt(deg)
    dinv_ref[...] = dv
    u = dv[:, 0:1] * jnp.dot(x_ref[...], w_ref[...], preferred_element_type=jnp.float32)
    u_ref[0] = u[:, :DHH]
    u_ref[1] = u[:, DHH:]


_tc_a = pl.pallas_call(
    _tc_a_body,
    grid=(N_PAD // BLK,),
    in_specs=[
        pl.BlockSpec((BLK, DIN), lambda i: (i, 0)),
        pl.BlockSpec((NC, BLK, 16), lambda i: (0, i, 0)),
        pl.BlockSpec((DIN, DH), lambda i: (0, 0)),
    ],
    out_specs=[
        pl.BlockSpec((NC, BLK, DHH), lambda i: (0, i, 0)),
        pl.BlockSpec((BLK, 16), lambda i: (i, 0)),
    ],
    out_shape=[
        jax.ShapeDtypeStruct((NC, N_PAD, DHH), jnp.float32),
        jax.ShapeDtypeStruct((N_PAD, 16), jnp.float32),
    ],
)


def _tc_b_body(acc_ref, dinv_ref, w_ref, b_ref, un_ref):
    dv = dinv_ref[...][:, 0:1]
    s = jnp.concatenate([acc_ref[0], acc_ref[1]], axis=1)
    t = jnp.maximum(dv * s + b_ref[...], 0.0)
    u = dv * jnp.dot(t, w_ref[...], preferred_element_type=jnp.float32)
    un_ref[0] = u[:, :DHH]
    un_ref[1] = u[:, DHH:]


_tc_b = pl.pallas_call(
    _tc_b_body,
    grid=(N_PAD // BLK,),
    in_specs=[
        pl.BlockSpec((NC, BLK, DHH), lambda i: (0, i, 0)),
        pl.BlockSpec((BLK, 16), lambda i: (i, 0)),
        pl.BlockSpec((DH, DH), lambda i: (0, 0)),
        pl.BlockSpec((1, DH), lambda i: (0, 0)),
    ],
    out_specs=pl.BlockSpec((NC, BLK, DHH), lambda i: (0, i, 0)),
    out_shape=jax.ShapeDtypeStruct((NC, N_PAD, DHH), jnp.float32),
)


def _tc_f_body(acc_ref, dinv_ref, b_ref, batch_ref, wlin_ref, blin_ref, out_ref, sums_ref, cnt_ref):
    i = pl.program_id(0)

    @pl.when(i == 0)
    def _():
        sums_ref[...] = jnp.zeros_like(sums_ref)
        cnt_ref[...] = jnp.zeros_like(cnt_ref)

    dv = dinv_ref[...][:, 0:1]
    s = jnp.concatenate([acc_ref[0], acc_ref[1]], axis=1)
    t = jnp.maximum(dv * s + b_ref[...], 0.0)
    p = (batch_ref[...] == lax.broadcasted_iota(jnp.int32, (BLK, G), 1)).astype(
        jnp.float32
    )
    sums_ref[...] += lax.dot_general(
        p, t, (((0,), (0,)), ((), ())), preferred_element_type=jnp.float32
    )
    cnt_ref[...] += lax.dot_general(
        p,
        jnp.ones((BLK, DH), jnp.float32),
        (((0,), (0,)), ((), ())),
        preferred_element_type=jnp.float32,
    )

    @pl.when(i == pl.num_programs(0) - 1)
    def _():
        pooled = sums_ref[...] / jnp.maximum(cnt_ref[...], 1.0)
        out_ref[...] = (
            jnp.dot(pooled, wlin_ref[...], preferred_element_type=jnp.float32)
            + blin_ref[...]
        )


_tc_f = pl.pallas_call(
    _tc_f_body,
    grid=(N_PAD // BLK,),
    in_specs=[
        pl.BlockSpec((NC, BLK, DHH), lambda i: (0, i, 0)),
        pl.BlockSpec((BLK, 16), lambda i: (i, 0)),
        pl.BlockSpec((1, DH), lambda i: (0, 0)),
        pl.BlockSpec((BLK, 1), lambda i: (i, 0)),
        pl.BlockSpec((DH, DOUT), lambda i: (0, 0)),
        pl.BlockSpec((1, DOUT), lambda i: (0, 0)),
    ],
    out_specs=pl.BlockSpec((G, DOUT), lambda i: (0, 0)),
    out_shape=jax.ShapeDtypeStruct((G, DOUT), jnp.float32),
    scratch_shapes=[
        pltpu.VMEM((G, DH), jnp.float32),
        pltpu.VMEM((G, DH), jnp.float32),
    ],
)


def kernel(x, edge_index, batch, W1, b1, Wh, bh, Wlin, blin):
    src = edge_index[0]
    dst = edge_index[1]
    srcs = jnp.full((E_PAD,), PAD_ROW, jnp.int32).at[:E].set(src).reshape(NS, CHE, K)
    dsts = jnp.full((E_PAD,), PAD_ROW, jnp.int32).at[:E].set(dst).reshape(NS, CHE, K)
    zeros16 = jnp.zeros((N_PAD, 16), jnp.float32)
    ones16 = jnp.ones((K, 16), jnp.float32)
    x_p = jnp.zeros((N_PAD, DIN), jnp.float32).at[:N].set(x)
    batch_p = jnp.full((N_PAD, 1), G, jnp.int32).at[:N, 0].set(batch)

    dega = _sc_deg(dsts, ones16, zeros16)
    u, dinv = _tc_a(x_p, dega, W1)
    biases = [b1] + [bh[i] for i in range(8)]
    for i in range(9):
        acc = _sc_edge(u, srcs, dsts)
        u = _tc_b(acc, dinv, Wh[i], biases[i].reshape(1, DH))
    acc = _sc_edge(u, srcs, dsts)
    return _tc_f(
        acc, dinv, bh[8].reshape(1, DH), batch_p, Wlin, blin.reshape(1, DOUT)
    )
